# baseline (speedup 1.0000x reference)
import math

import jax
import jax.numpy as jnp
from jax.experimental import pallas as pl
from jax.experimental.pallas import tpu as pltpu

_B, _N, _H, _L, _K = 8, 100, 128, 2, 10
_TI = 20
_NT = _N // _TI
_R = _TI * _N
_F32 = jnp.float32


def _dgt(x, w):
    return jax.lax.dot_general(x, w, (((1,), (1,)), ((), ())),
                               preferred_element_type=_F32)


def _ln(x, g, b):
    m = jnp.mean(x, axis=-1, keepdims=True)
    d = x - m
    v = jnp.mean(d * d, axis=-1, keepdims=True)
    return d / jnp.sqrt(v + 1e-5) * g + b


def _ln_mxu(x, g, b):
    j = jnp.full((_H, _H), 1.0 / _H, _F32)
    m = jnp.dot(x, j, preferred_element_type=_F32)
    s2 = jnp.dot(x * x, j, preferred_element_type=_F32)
    return (x - m) * jax.lax.rsqrt(s2 - m * m + 1e-5) * g + b


def _fused_kernel(dist_ref, coor_ref, info_ref,
                  w1_ref, w2_ref, w3_ref, wi_ref,
                  wq0_ref, wk0_ref, wna0_ref, wv0_ref, wnc0_ref, wvn0_ref,
                  we20_ref, we30_ref,
                  wq1_ref, wk1_ref, wna1_ref, wv1_ref, wnc1_ref, wvn1_ref,
                  we21_ref, we31_ref,
                  wie_ref,
                  wea10_ref, wea0_ref, wec0_ref, wve0_ref,
                  wea11_ref, wea1_ref, wec1_ref, wve1_ref,
                  nvec_ref, evec_ref,
                  hn_ref, out_ref,
                  adj_scr, e2a_scr, e3a_scr, e2b_scr, e3b_scr):
    t = pl.program_id(1)

    @pl.when(t == 0)
    def _prep():
        w = dist_ref[0]
        col = jax.lax.broadcasted_iota(jnp.int32, (_N, _N), 1)
        row = jax.lax.broadcasted_iota(jnp.int32, (_N, _N), 0)
        adj = jnp.zeros((_N, _N), _F32)
        msk = jnp.zeros((_N, _N), _F32)
        for k in range(_K + 1):
            m = jnp.min(w, axis=-1, keepdims=True)
            cand = w == m
            idx = jnp.min(jnp.where(cand, col, jnp.int32(2**30)), axis=-1,
                          keepdims=True)
            hit = col == idx
            adj = adj + hit.astype(_F32)
            if k >= 1:
                msk = msk + hit.astype(_F32)
            w = jnp.where(hit, _F32(jnp.inf), w)
        adj_scr[...] = jnp.where(row == col, _F32(-1.0), adj)

        coor = coor_ref[0]
        info = info_ref[0]
        x0 = jnp.maximum(_dgt(coor, w1_ref[...]) + nvec_ref[0], 0.0)
        xi = jnp.maximum(
            jnp.concatenate([_dgt(coor, w2_ref[...]),
                             _dgt(info, w3_ref[...])], axis=-1)
            + nvec_ref[1], 0.0)
        rid = jax.lax.broadcasted_iota(jnp.int32, (_N, _H), 0)
        x = jnp.where(rid == 0, x0, xi)
        h = _dgt(x, wi_ref[...]) + nvec_ref[2]
        mats = ((wq0_ref, wk0_ref, wna0_ref, wv0_ref, wnc0_ref, wvn0_ref,
                 we20_ref, we30_ref),
                (wq1_ref, wk1_ref, wna1_ref, wv1_ref, wnc1_ref, wvn1_ref,
                 we21_ref, we31_ref))
        e2scrs = (e2a_scr, e2b_scr)
        e3scrs = (e3a_scr, e3b_scr)
        for l in range(_L):
            wq, wk, wna, wv, wnc, wvn, we2, we3 = mats[l]
            vb = 3 + 12 * l
            q = _dgt(h, wq[...]) + nvec_ref[vb + 0]
            k_ = _dgt(h, wk[...]) + nvec_ref[vb + 1]
            s = _dgt(q, k_) * (1.0 / math.sqrt(_H))
            s = jnp.where(msk > 0.5, s, _F32(-1e30))
            smax = jnp.max(s, axis=-1, keepdims=True)
            e = jnp.exp(s - smax)
            att = e / jnp.sum(e, axis=-1, keepdims=True)
            wnav = jnp.dot(wna[...], wv[...], preferred_element_type=_F32)
            vprime = _dgt(h, wnav)
            battn = _dgt(nvec_ref[vb + 2], wna[...]) + nvec_ref[vb + 3]
            hagg = h + jnp.maximum(
                jnp.dot(att, vprime, preferred_element_type=_F32) + battn,
                0.0)
            hagg = _ln(hagg, nvec_ref[vb + 4], nvec_ref[vb + 5])
            wcn = jnp.dot(wvn[:, :_H], wnc[...], preferred_element_type=_F32)
            bcn = _dgt(nvec_ref[vb + 6], wvn[:, :_H]) + nvec_ref[vb + 7]
            hcom = hagg + jnp.maximum(
                _dgt(h, wcn) + _dgt(hagg, wvn[:, _H:]) + bcn, 0.0)
            hn_next = _ln(hcom, nvec_ref[vb + 8], nvec_ref[vb + 9])
            e2scrs[l][...] = _dgt(h, we2[...]) + nvec_ref[vb + 10]
            e3scrs[l][...] = _dgt(h, we3[...]) + nvec_ref[vb + 11]
            h = hn_next
        hn_ref[0] = h

    rows = pl.ds(t * _TI, _TI)
    d3 = dist_ref[0, rows, :][:, :, None]
    a3 = adj_scr[rows, :][:, :, None]
    u = evec_ref[0][None]
    z = evec_ref[1][None]
    bb = evec_ref[2][None]
    y = jnp.maximum(d3 * u + a3 * z + bb, 0.0).reshape(_R, _H)
    he = _dgt(y, wie_ref[...]) + evec_ref[3]
    emats = ((wea10_ref, wea0_ref, wec0_ref, wve0_ref),
             (wea11_ref, wea1_ref, wec1_ref, wve1_ref))
    e2scrs = (e2a_scr, e2b_scr)
    e3scrs = (e3a_scr, e3b_scr)
    for l in range(_L):
        wea1, wea, wec, wve = emats[l]
        vb = 4 + 7 * l
        e1 = _dgt(he, wea1[...])
        e2 = e2scrs[l][rows, :]
        e3 = e3scrs[l][...]
        s = (e1.reshape(_TI, _N, _H) + e2[:, None, :]
             + e3[None, :, :]).reshape(_R, _H)
        tt = jnp.maximum(_dgt(s, wea[...]) + evec_ref[vb + 0], 0.0)
        hagg = _ln_mxu(he + tt, evec_ref[vb + 1], evec_ref[vb + 2])
        wce = jnp.dot(wve[:, :_H], wec[...], preferred_element_type=_F32)
        bce = _dgt(evec_ref[vb + 3], wve[:, :_H]) + evec_ref[vb + 4]
        hcom = hagg + jnp.maximum(
            _dgt(he, wce) + _dgt(hagg, wve[:, _H:]) + bce, 0.0)
        he = _ln_mxu(hcom, evec_ref[vb + 5], evec_ref[vb + 6])
    out_ref[0] = he.reshape(_TI, _N, _H)


def _run(params, n_coor, n_info, dist, interpret=False):
    p = params
    lp0, lp1 = p["layers"]
    half = _H // 2
    zeros_h = jnp.zeros((half,), _F32)

    nvecs = [p["W1"]["b"],
             jnp.concatenate([p["W2"]["b"], p["W3"]["b"]]),
             p["init_n"]["b"]]
    nmats = [p["W1"]["w"], p["W2"]["w"], p["W3"]["w"], p["init_n"]["w"]]
    for lp in (lp0, lp1):
        nmats += [lp["attn_q"]["w"], lp["attn_k"]["w"],
                  lp["W_node_agg"]["w"], lp["attn_v"]["w"],
                  lp["V_node_com"]["w"], lp["V_node"]["w"],
                  lp["W_edge_agg_2"]["w"], lp["W_edge_agg_3"]["w"]]
        nvecs += [lp["attn_q"]["b"], lp["attn_k"]["b"],
                  lp["attn_v"]["b"], lp["W_node_agg"]["b"],
                  lp["ln_na"]["g"], lp["ln_na"]["b"],
                  lp["V_node_com"]["b"], lp["V_node"]["b"],
                  lp["ln_nc"]["g"], lp["ln_nc"]["b"],
                  lp["W_edge_agg_2"]["b"] + lp["W_edge_agg_1"]["b"],
                  lp["W_edge_agg_3"]["b"]]
    nvecs = jnp.stack(nvecs)[:, None, :]

    evecs = [jnp.concatenate([p["W4"]["w"][:, 0], zeros_h]),
             jnp.concatenate([zeros_h, p["W5"]["w"][:, 0]]),
             jnp.concatenate([p["W4"]["b"], p["W5"]["b"]]),
             p["init_e"]["b"]]
    emats = [p["init_e"]["w"]]
    for lp in (lp0, lp1):
        emats += [lp["W_edge_agg_1"]["w"], lp["W_edge_agg"]["w"],
                  lp["V_edge_com"]["w"], lp["V_edge"]["w"]]
        evecs += [lp["W_edge_agg"]["b"],
                  lp["ln_ea"]["g"], lp["ln_ea"]["b"],
                  lp["V_edge_com"]["b"], lp["V_edge"]["b"],
                  lp["ln_ec"]["g"], lp["ln_ec"]["b"]]
    evecs = jnp.stack(evecs)[:, None, :]

    mspec = [pl.BlockSpec(m.shape, lambda b, t: (0, 0))
             for m in nmats + emats]
    hn, he = pl.pallas_call(
        _fused_kernel,
        grid=(_B, _NT),
        in_specs=[
            pl.BlockSpec((1, _N, _N), lambda b, t: (b, 0, 0)),
            pl.BlockSpec((1, _N, 2), lambda b, t: (b, 0, 0)),
            pl.BlockSpec((1, _N, 3), lambda b, t: (b, 0, 0)),
            *mspec,
            pl.BlockSpec(nvecs.shape, lambda b, t: (0, 0, 0)),
            pl.BlockSpec(evecs.shape, lambda b, t: (0, 0, 0)),
        ],
        out_specs=[
            pl.BlockSpec((1, _N, _H), lambda b, t: (b, 0, 0)),
            pl.BlockSpec((1, _TI, _N, _H), lambda b, t: (b, t, 0, 0)),
        ],
        out_shape=[
            jax.ShapeDtypeStruct((_B, _N, _H), _F32),
            jax.ShapeDtypeStruct((_B, _N, _N, _H), _F32),
        ],
        scratch_shapes=[
            pltpu.VMEM((_N, _N), _F32),
            pltpu.VMEM((_N, _H), _F32), pltpu.VMEM((_N, _H), _F32),
            pltpu.VMEM((_N, _H), _F32), pltpu.VMEM((_N, _H), _F32),
        ],
        interpret=interpret,
    )(dist, n_coor, n_info, *nmats, *emats, nvecs, evecs)

    return hn, he


def kernel(params, n_coor, n_info, dist):
    return _run(params, n_coor, n_info, dist)
